# baseline (speedup 1.0000x reference)
import functools

import jax
import jax.numpy as jnp
from jax import lax
from jax.experimental import pallas as pl
from jax.experimental.pallas import tpu as pltpu
from jax.experimental.pallas import tpu_sc as plsc

N = 100
E = 4000
KPOOL = 50
NCLUST = 50
NEG_SLOPE = 0.01

NC, NS, L = 2, 16, 16
NTILES = NC * NS
EPAD = 4096
CHUNK = EPAD // NTILES
ACC_ROWS = 2 * N
ACC_LANES = 128
ACC_FLAT = ACC_ROWS * ACC_LANES

_HI = lax.Precision.HIGHEST
_DF = lax.Precision.DEFAULT


def _dot(a, b, dims, prec):
    return lax.dot_general(a, b, (dims, ((), ())),
                           precision=prec, preferred_element_type=jnp.float32)


def _mm(a, b, prec):
    return _dot(a, b, ((1,), (0,)), prec)


def _leaky(v):
    return jnp.where(v >= 0, v, NEG_SLOPE * v)


def _softmax(v):
    m = jnp.max(v, axis=-1, keepdims=True)
    e = jnp.exp(v - m)
    return e / jnp.sum(e, axis=-1, keepdims=True)


@functools.lru_cache(maxsize=1)
def _densify_sc():
    mesh = plsc.VectorSubcoreMesh(core_axis_name="c", subcore_axis_name="s")
    epc = EPAD // NC

    @functools.partial(
        pl.kernel, mesh=mesh,
        out_type=jax.ShapeDtypeStruct((NC, ACC_FLAT), jnp.float32),
        scratch_types=[
            pltpu.VMEM((epc,), jnp.int32),
            pltpu.VMEM((epc,), jnp.int32),
            pltpu.VMEM((epc,), jnp.float32),
            pltpu.VMEM((epc,), jnp.float32),
            pltpu.VMEM((epc,), jnp.int32),
            pltpu.VMEM((epc,), jnp.int32),
            pltpu.VMEM_SHARED((ACC_FLAT,), jnp.float32),
        ],
    )
    def dens(row_hbm, col_hbm, ea_hbm, ones_hbm, zero_hbm, out_hbm,
             row_v, col_v, ea_v, ones_v, idxw_v, idxc_v, shared):
        cid = lax.axis_index("c")
        sid = lax.axis_index("s")

        @pl.when(sid == 0)
        def _work():
            base = cid * epc
            pltpu.sync_copy(zero_hbm, shared)
            pltpu.sync_copy(row_hbm.at[pl.ds(base, epc)], row_v)
            pltpu.sync_copy(col_hbm.at[pl.ds(base, epc)], col_v)
            pltpu.sync_copy(ea_hbm.at[pl.ds(base, epc)], ea_v)
            pltpu.sync_copy(ones_hbm.at[pl.ds(base, epc)], ones_v)
            for v in range(epc // L):
                sl = pl.ds(v * L, L)
                flat = col_v[sl] * ACC_LANES + row_v[sl]
                idxw_v[sl] = flat
                idxc_v[sl] = flat + N * ACC_LANES
            pltpu.sync_copy(ea_v, shared.at[idxw_v], add=True)
            pltpu.sync_copy(ones_v, shared.at[idxc_v], add=True)
            pltpu.sync_copy(shared, out_hbm.at[cid])

    return dens


def _fwd_kernel(acc_ref, x_ref,
                wl1_ref, bl1_ref, wr1_ref, br1_ref,
                wl2_ref, bl2_ref, wr2_ref, br2_ref,
                wg1_ref, bg1_ref, wrel_ref, brel_ref, wroot_ref,
                wc0_ref, wc1_ref, wc2_ref, bc_ref,
                out_ref):
    f32 = jnp.float32
    acc = (acc_ref[0] + acc_ref[1]).reshape(ACC_ROWS, ACC_LANES)
    ci0 = lax.broadcasted_iota(jnp.int32, (N, N), 0)
    ri0 = lax.broadcasted_iota(jnp.int32, (N, N), 1)
    eye00 = ((ci0 == 0) & (ri0 == 0)).astype(f32)
    a_w = acc[0:N, 0:N]
    bc_mat = acc[N:2 * N, 0:N] - (EPAD - E) * eye00

    eye = (lax.broadcasted_iota(jnp.int32, (N, N), 0)
           == lax.broadcasted_iota(jnp.int32, (N, N), 1)).astype(f32)
    a_f = a_w + eye
    ci = lax.broadcasted_iota(jnp.int32, (N, N), 0)
    ri = lax.broadcasted_iota(jnp.int32, (N, N), 1)
    in_l = (ci < KPOOL) & (ri < KPOOL)
    in_r = (ci >= KPOOL) & (ri >= KPOOL)
    a_l = jnp.where(in_l, a_w, 0.0) + eye
    a_r = jnp.where(in_r, a_w, 0.0) + eye

    def inv_sqrt_deg(a):
        deg = jnp.sum(a, axis=1, keepdims=True)
        return lax.rsqrt(deg)

    dis_f = inv_sqrt_deg(a_f)
    dis_l = inv_sqrt_deg(a_l)
    dis_r = inv_sqrt_deg(a_r)

    def gcn(xw, a, dis, b):
        return dis * _mm(a, dis * xw, _HI) + b

    x = x_ref[...]
    riota64 = lax.broadcasted_iota(jnp.int32, (N, 64), 0)
    riota20 = lax.broadcasted_iota(jnp.int32, (N, 20), 0)

    hl = _leaky(gcn(_mm(x, wl1_ref[...], _DF), a_l, dis_l, bl1_ref[...]))
    hr = _leaky(gcn(_mm(x, wr1_ref[...], _DF), a_r, dis_r, br1_ref[...]))
    h1 = jnp.where(riota64 < KPOOL, hl, hr)

    hl2 = _leaky(gcn(_mm(h1, wl2_ref[...], _DF), a_l, dis_l, bl2_ref[...]))
    hr2 = _leaky(gcn(_mm(h1, wr2_ref[...], _DF), a_r, dis_r, br2_ref[...]))
    h2a = jnp.where(riota20 < KPOOL, hl2, hr2)

    h2 = _leaky(gcn(_mm(h2a, wg1_ref[...], _DF), a_f, dis_f, bg1_ref[...]))

    agg = _mm(bc_mat, h2, _HI)
    score = jnp.tanh(_mm(agg, wrel_ref[...], _DF) + brel_ref[...]
                     + _mm(h2, wroot_ref[...], _DF))

    score_row = _dot(score, eye, ((0,), (0,)), _HI)
    ri = lax.broadcasted_iota(jnp.int32, (N, N), 0)
    rj = lax.broadcasted_iota(jnp.int32, (N, N), 1)
    beats = ((score_row > score) |
             ((score_row == score) & (rj < ri))).astype(f32)
    rank = jnp.sum(beats, axis=1, keepdims=True)
    rank_row = _dot(rank, eye, ((0,), (0,)), _HI)

    piota = lax.broadcasted_iota(jnp.int32, (KPOOL, N), 0).astype(f32)
    perm_mat = (rank_row == piota).astype(f32)

    vals = _mm(perm_mat, score, _HI)
    x_pool = _mm(perm_mat, h2, _HI) * vals

    craw = _dot(_mm(perm_mat, bc_mat, _HI), perm_mat, ((1,), (1,)), _HI)
    deg_c = jnp.sum(craw, axis=1, keepdims=True)
    dis_c = jnp.where(deg_c > 0, lax.rsqrt(jnp.where(deg_c > 0, deg_c, 1.0)), 0.0)

    def prop_top(z):
        return -(dis_c * _mm(craw, dis_c * z, _HI))

    h2_top = h2[0:KPOOL, :]
    h2_bot = h2[KPOOL:N, :]
    t1_top = prop_top(h2_top)
    t2_top = 2.0 * prop_top(t1_top) - h2_top
    zeros_bot = jnp.zeros((N - KPOOL, 20), f32)
    tx1 = jnp.concatenate([t1_top, zeros_bot], axis=0)
    tx2 = jnp.concatenate([t2_top, -h2_bot], axis=0)

    cheb = (_mm(h2, wc0_ref[...], _DF) + _mm(tx1, wc1_ref[...], _DF)
            + _mm(tx2, wc2_ref[...], _DF) + bc_ref[...])
    ass = _softmax(cheb)
    s = _softmax(ass)

    h_coarse = _dot(s, h2, ((0,), (0,)), _DF)

    kept = (rank < KPOOL).astype(f32)
    tri = (rj < ri).astype(f32)
    cum_excl = _mm(tri, kept, _HI)
    cum_row = _dot(cum_excl, eye, ((0,), (0,)), _HI)
    kept_row = rank_row < KPOOL
    qiota = lax.broadcasted_iota(jnp.int32, (KPOOL, N), 0).astype(f32)
    q_mat = ((cum_row == qiota) & kept_row).astype(f32)

    inter = _mm(q_mat, ass, _HI)
    h1_out = _mm(inter, h_coarse, _DF)
    out_ref[...] = x_pool + h1_out


def kernel(x, edge_index, edge_attr, adj, W_l1, b_l1, W_r1, b_r1, W_l2, b_l2,
           W_r2, b_r2, W_g1, b_g1, W_rel, b_rel, W_root, W_c0, W_c1, W_c2, b_c,
           interpret=False):
    del adj
    f32 = jnp.float32
    row = edge_index[0]
    col = edge_index[1]
    ea = edge_attr.astype(f32)
    if interpret:
        flat = jnp.zeros((ACC_ROWS, ACC_LANES), f32)
        flat = flat.at[col, row].add(ea).at[col + N, row].add(1.0)
        flat = flat.at[N, 0].add(float(EPAD - E))
        acc2 = jnp.stack([flat.reshape(-1), jnp.zeros((ACC_FLAT,), f32)])
    else:
        pad = EPAD - E
        acc2 = _densify_sc()(
            jnp.pad(row, (0, pad)), jnp.pad(col, (0, pad)),
            jnp.pad(ea, (0, pad)),
            jnp.ones((EPAD,), f32),
            jnp.zeros((ACC_FLAT,), f32))
    operands = (
        acc2, x,
        W_l1, b_l1.reshape(1, -1), W_r1, b_r1.reshape(1, -1),
        W_l2, b_l2.reshape(1, -1), W_r2, b_r2.reshape(1, -1),
        W_g1, b_g1.reshape(1, -1), W_rel, b_rel.reshape(1, 1), W_root,
        W_c0, W_c1, W_c2, b_c.reshape(1, -1),
    )
    h2_out = pl.pallas_call(
        _fwd_kernel,
        out_shape=jax.ShapeDtypeStruct((KPOOL, 20), jnp.float32),
        interpret=interpret,
    )(*operands)
    return h2_out.reshape(1, -1)


# speedup vs baseline: 1.0426x; 1.0426x over previous
import functools

import jax
import jax.numpy as jnp
from jax import lax
from jax.experimental import pallas as pl
from jax.experimental.pallas import tpu as pltpu
from jax.experimental.pallas import tpu_sc as plsc

N = 100
E = 4000
KPOOL = 50
NCLUST = 50
NEG_SLOPE = 0.01

NC, NS, L = 2, 16, 16
NTILES = NC * NS
EPAD = 4096
CHUNK = EPAD // NTILES
ACC_ROWS = 2 * N
ACC_LANES = 128
ACC_FLAT = ACC_ROWS * ACC_LANES

_HI = lax.Precision.HIGHEST
_DF = lax.Precision.DEFAULT


def _dot(a, b, dims, prec):
    return lax.dot_general(a, b, (dims, ((), ())),
                           precision=prec, preferred_element_type=jnp.float32)


def _mm(a, b, prec):
    return _dot(a, b, ((1,), (0,)), prec)


def _leaky(v):
    return jnp.where(v >= 0, v, NEG_SLOPE * v)


def _softmax(v):
    m = jnp.max(v, axis=-1, keepdims=True)
    e = jnp.exp(v - m)
    return e / jnp.sum(e, axis=-1, keepdims=True)


@functools.lru_cache(maxsize=1)
def _densify_sc():
    mesh = plsc.VectorSubcoreMesh(core_axis_name="c", subcore_axis_name="s")

    @functools.partial(
        pl.kernel, mesh=mesh,
        out_type=jax.ShapeDtypeStruct((NC, ACC_FLAT), jnp.float32),
        scratch_types=[
            pltpu.VMEM((CHUNK,), jnp.int32),
            pltpu.VMEM((CHUNK,), jnp.int32),
            pltpu.VMEM((CHUNK,), jnp.float32),
            pltpu.VMEM((CHUNK,), jnp.float32),
            pltpu.VMEM((CHUNK,), jnp.int32),
            pltpu.VMEM((CHUNK,), jnp.int32),
            pltpu.VMEM_SHARED((ACC_FLAT,), jnp.float32),
        ],
    )
    def dens(row_hbm, col_hbm, ea_hbm, ones_hbm, zero_hbm, out_hbm,
             row_v, col_v, ea_v, ones_v, idxw_v, idxc_v, shared):
        cid = lax.axis_index("c")
        sid = lax.axis_index("s")
        base = (cid * NS + sid) * CHUNK
        pltpu.sync_copy(row_hbm.at[pl.ds(base, CHUNK)], row_v)
        pltpu.sync_copy(col_hbm.at[pl.ds(base, CHUNK)], col_v)
        pltpu.sync_copy(ea_hbm.at[pl.ds(base, CHUNK)], ea_v)
        pltpu.sync_copy(ones_hbm.at[pl.ds(base, CHUNK)], ones_v)
        for v in range(CHUNK // L):
            sl = pl.ds(v * L, L)
            flat = col_v[sl] * ACC_LANES + row_v[sl]
            idxw_v[sl] = flat
            idxc_v[sl] = flat + N * ACC_LANES

        @pl.when(sid == 0)
        def _zero_shared():
            pltpu.sync_copy(zero_hbm, shared)

        plsc.subcore_barrier()
        pltpu.sync_copy(ea_v, shared.at[idxw_v], add=True)
        pltpu.sync_copy(ones_v, shared.at[idxc_v], add=True)
        plsc.subcore_barrier()

        @pl.when(sid == 0)
        def _publish():
            pltpu.sync_copy(shared, out_hbm.at[cid])

    return dens


def _fwd_kernel(acc_ref, x_ref,
                wl1_ref, bl1_ref, wr1_ref, br1_ref,
                wl2_ref, bl2_ref, wr2_ref, br2_ref,
                wg1_ref, bg1_ref, wrel_ref, brel_ref, wroot_ref,
                wc0_ref, wc1_ref, wc2_ref, bc_ref,
                out_ref):
    f32 = jnp.float32
    acc = (acc_ref[0] + acc_ref[1]).reshape(ACC_ROWS, ACC_LANES)
    ci0 = lax.broadcasted_iota(jnp.int32, (N, N), 0)
    ri0 = lax.broadcasted_iota(jnp.int32, (N, N), 1)
    eye00 = ((ci0 == 0) & (ri0 == 0)).astype(f32)
    a_w = acc[0:N, 0:N]
    bc_mat = acc[N:2 * N, 0:N] - (EPAD - E) * eye00

    eye = (lax.broadcasted_iota(jnp.int32, (N, N), 0)
           == lax.broadcasted_iota(jnp.int32, (N, N), 1)).astype(f32)
    a_f = a_w + eye
    ci = lax.broadcasted_iota(jnp.int32, (N, N), 0)
    ri = lax.broadcasted_iota(jnp.int32, (N, N), 1)
    in_l = (ci < KPOOL) & (ri < KPOOL)
    in_r = (ci >= KPOOL) & (ri >= KPOOL)
    a_l = jnp.where(in_l, a_w, 0.0) + eye
    a_r = jnp.where(in_r, a_w, 0.0) + eye

    def inv_sqrt_deg(a):
        deg = jnp.sum(a, axis=1, keepdims=True)
        return lax.rsqrt(deg)

    dis_f = inv_sqrt_deg(a_f)
    dis_l = inv_sqrt_deg(a_l)
    dis_r = inv_sqrt_deg(a_r)

    def gcn(xw, a, dis, b):
        return dis * _mm(a, dis * xw, _HI) + b

    x = x_ref[...]
    riota64 = lax.broadcasted_iota(jnp.int32, (N, 64), 0)
    riota20 = lax.broadcasted_iota(jnp.int32, (N, 20), 0)

    hl = _leaky(gcn(_mm(x, wl1_ref[...], _DF), a_l, dis_l, bl1_ref[...]))
    hr = _leaky(gcn(_mm(x, wr1_ref[...], _DF), a_r, dis_r, br1_ref[...]))
    h1 = jnp.where(riota64 < KPOOL, hl, hr)

    hl2 = _leaky(gcn(_mm(h1, wl2_ref[...], _DF), a_l, dis_l, bl2_ref[...]))
    hr2 = _leaky(gcn(_mm(h1, wr2_ref[...], _DF), a_r, dis_r, br2_ref[...]))
    h2a = jnp.where(riota20 < KPOOL, hl2, hr2)

    h2 = _leaky(gcn(_mm(h2a, wg1_ref[...], _DF), a_f, dis_f, bg1_ref[...]))

    agg = _mm(bc_mat, h2, _HI)
    score = jnp.tanh(_mm(agg, wrel_ref[...], _DF) + brel_ref[...]
                     + _mm(h2, wroot_ref[...], _DF))

    score_row = _dot(score, eye, ((0,), (0,)), _HI)
    ri = lax.broadcasted_iota(jnp.int32, (N, N), 0)
    rj = lax.broadcasted_iota(jnp.int32, (N, N), 1)
    beats = ((score_row > score) |
             ((score_row == score) & (rj < ri))).astype(f32)
    rank = jnp.sum(beats, axis=1, keepdims=True)
    rank_row = _dot(rank, eye, ((0,), (0,)), _HI)

    piota = lax.broadcasted_iota(jnp.int32, (KPOOL, N), 0).astype(f32)
    perm_mat = (rank_row == piota).astype(f32)

    vals = _mm(perm_mat, score, _HI)
    x_pool = _mm(perm_mat, h2, _HI) * vals

    craw = _dot(_mm(perm_mat, bc_mat, _HI), perm_mat, ((1,), (1,)), _HI)
    deg_c = jnp.sum(craw, axis=1, keepdims=True)
    dis_c = jnp.where(deg_c > 0, lax.rsqrt(jnp.where(deg_c > 0, deg_c, 1.0)), 0.0)

    def prop_top(z):
        return -(dis_c * _mm(craw, dis_c * z, _HI))

    h2_top = h2[0:KPOOL, :]
    h2_bot = h2[KPOOL:N, :]
    t1_top = prop_top(h2_top)
    t2_top = 2.0 * prop_top(t1_top) - h2_top
    zeros_bot = jnp.zeros((N - KPOOL, 20), f32)
    tx1 = jnp.concatenate([t1_top, zeros_bot], axis=0)
    tx2 = jnp.concatenate([t2_top, -h2_bot], axis=0)

    cheb = (_mm(h2, wc0_ref[...], _DF) + _mm(tx1, wc1_ref[...], _DF)
            + _mm(tx2, wc2_ref[...], _DF) + bc_ref[...])
    ass = _softmax(cheb)
    s = _softmax(ass)

    h_coarse = _dot(s, h2, ((0,), (0,)), _DF)

    kept = (rank < KPOOL).astype(f32)
    tri = (rj < ri).astype(f32)
    cum_excl = _mm(tri, kept, _HI)
    cum_row = _dot(cum_excl, eye, ((0,), (0,)), _HI)
    kept_row = rank_row < KPOOL
    qiota = lax.broadcasted_iota(jnp.int32, (KPOOL, N), 0).astype(f32)
    q_mat = ((cum_row == qiota) & kept_row).astype(f32)

    inter = _mm(q_mat, ass, _HI)
    h1_out = _mm(inter, h_coarse, _DF)
    out_ref[...] = x_pool + h1_out


def kernel(x, edge_index, edge_attr, adj, W_l1, b_l1, W_r1, b_r1, W_l2, b_l2,
           W_r2, b_r2, W_g1, b_g1, W_rel, b_rel, W_root, W_c0, W_c1, W_c2, b_c,
           interpret=False):
    del adj
    f32 = jnp.float32
    row = edge_index[0]
    col = edge_index[1]
    ea = edge_attr.astype(f32)
    if interpret:
        flat = jnp.zeros((ACC_ROWS, ACC_LANES), f32)
        flat = flat.at[col, row].add(ea).at[col + N, row].add(1.0)
        flat = flat.at[N, 0].add(float(EPAD - E))
        acc2 = jnp.stack([flat.reshape(-1), jnp.zeros((ACC_FLAT,), f32)])
    else:
        pad = EPAD - E
        acc2 = _densify_sc()(
            jnp.pad(row, (0, pad)), jnp.pad(col, (0, pad)),
            jnp.pad(ea, (0, pad)),
            jnp.ones((EPAD,), f32),
            jnp.zeros((ACC_FLAT,), f32))
    operands = (
        acc2, x,
        W_l1, b_l1.reshape(1, -1), W_r1, b_r1.reshape(1, -1),
        W_l2, b_l2.reshape(1, -1), W_r2, b_r2.reshape(1, -1),
        W_g1, b_g1.reshape(1, -1), W_rel, b_rel.reshape(1, 1), W_root,
        W_c0, W_c1, W_c2, b_c.reshape(1, -1),
    )
    h2_out = pl.pallas_call(
        _fwd_kernel,
        out_shape=jax.ShapeDtypeStruct((KPOOL, 20), jnp.float32),
        interpret=interpret,
    )(*operands)
    return h2_out.reshape(1, -1)


# speedup vs baseline: 1.0467x; 1.0040x over previous
import functools

import jax
import jax.numpy as jnp
from jax import lax
from jax.experimental import pallas as pl
from jax.experimental.pallas import tpu as pltpu
from jax.experimental.pallas import tpu_sc as plsc

N = 100
E = 4000
KPOOL = 50
NCLUST = 50
NEG_SLOPE = 0.01

NC, NS, L = 2, 16, 16
NTILES = NC * NS
EPAD = 4096
CHUNK = EPAD // NTILES
ACC_ROWS = 2 * N
ACC_LANES = 128
ACC_FLAT = ACC_ROWS * ACC_LANES

_HI = lax.Precision.HIGHEST
_DF = lax.Precision.DEFAULT


def _dot(a, b, dims, prec):
    return lax.dot_general(a, b, (dims, ((), ())),
                           precision=prec, preferred_element_type=jnp.float32)


def _mm(a, b, prec):
    return _dot(a, b, ((1,), (0,)), prec)


def _leaky(v):
    return jnp.where(v >= 0, v, NEG_SLOPE * v)


def _softmax(v):
    m = jnp.max(v, axis=-1, keepdims=True)
    e = jnp.exp(v - m)
    return e / jnp.sum(e, axis=-1, keepdims=True)


@functools.lru_cache(maxsize=1)
def _densify_sc():
    mesh = plsc.VectorSubcoreMesh(core_axis_name="c", subcore_axis_name="s")

    @functools.partial(
        pl.kernel, mesh=mesh,
        out_type=jax.ShapeDtypeStruct((NC, ACC_FLAT), jnp.float32),
        scratch_types=[
            pltpu.VMEM((CHUNK,), jnp.int32),
            pltpu.VMEM((CHUNK,), jnp.int32),
            pltpu.VMEM((CHUNK,), jnp.float32),
            pltpu.VMEM((CHUNK,), jnp.float32),
            pltpu.VMEM((CHUNK,), jnp.int32),
            pltpu.VMEM((CHUNK,), jnp.int32),
            pltpu.VMEM_SHARED((ACC_FLAT,), jnp.float32),
        ],
    )
    def dens(row_hbm, col_hbm, ea_hbm, ones_hbm, zero_hbm, out_hbm,
             row_v, col_v, ea_v, ones_v, idxw_v, idxc_v, shared):
        cid = lax.axis_index("c")
        sid = lax.axis_index("s")
        base = (cid * NS + sid) * CHUNK
        pltpu.sync_copy(row_hbm.at[pl.ds(base, CHUNK)], row_v)
        pltpu.sync_copy(col_hbm.at[pl.ds(base, CHUNK)], col_v)
        pltpu.sync_copy(ea_hbm.at[pl.ds(base, CHUNK)], ea_v)
        pltpu.sync_copy(ones_hbm.at[pl.ds(base, CHUNK)], ones_v)
        for v in range(CHUNK // L):
            sl = pl.ds(v * L, L)
            flat = col_v[sl] * ACC_LANES + row_v[sl]
            idxw_v[sl] = flat
            idxc_v[sl] = flat + N * ACC_LANES

        @pl.when(sid == 0)
        def _zero_shared():
            pltpu.sync_copy(zero_hbm, shared)

        plsc.subcore_barrier()
        pltpu.sync_copy(ea_v, shared.at[idxw_v], add=True)
        pltpu.sync_copy(ones_v, shared.at[idxc_v], add=True)
        plsc.subcore_barrier()

        @pl.when(sid == 0)
        def _publish():
            pltpu.sync_copy(shared, out_hbm.at[cid])

    return dens


def _fwd_kernel(acc_ref, x_ref,
                wl1_ref, bl1_ref, wr1_ref, br1_ref,
                wl2_ref, bl2_ref, wr2_ref, br2_ref,
                wg1_ref, bg1_ref, wrel_ref, brel_ref, wroot_ref,
                wc0_ref, wc1_ref, wc2_ref, bc_ref,
                out_ref):
    f32 = jnp.float32
    acc = (acc_ref[0] + acc_ref[1]).reshape(ACC_ROWS, ACC_LANES)
    ci0 = lax.broadcasted_iota(jnp.int32, (N, N), 0)
    ri0 = lax.broadcasted_iota(jnp.int32, (N, N), 1)
    eye00 = ((ci0 == 0) & (ri0 == 0)).astype(f32)
    a_w = acc[0:N, 0:N]
    bc_mat = acc[N:2 * N, 0:N] - (EPAD - E) * eye00

    eye = (lax.broadcasted_iota(jnp.int32, (N, N), 0)
           == lax.broadcasted_iota(jnp.int32, (N, N), 1)).astype(f32)
    a_f = a_w + eye
    ci = lax.broadcasted_iota(jnp.int32, (N, N), 0)
    ri = lax.broadcasted_iota(jnp.int32, (N, N), 1)
    in_l = (ci < KPOOL) & (ri < KPOOL)
    in_r = (ci >= KPOOL) & (ri >= KPOOL)
    a_l = jnp.where(in_l, a_w, 0.0) + eye
    a_r = jnp.where(in_r, a_w, 0.0) + eye

    def inv_sqrt_deg(a):
        deg = jnp.sum(a, axis=1, keepdims=True)
        return lax.rsqrt(deg)

    dis_f = inv_sqrt_deg(a_f)
    dis_l = inv_sqrt_deg(a_l)
    dis_r = inv_sqrt_deg(a_r)

    def gcn(xw, a, dis, b):
        return dis * _mm(a, dis * xw, _HI) + b

    x = x_ref[...]
    riota64 = lax.broadcasted_iota(jnp.int32, (N, 64), 0)
    riota20 = lax.broadcasted_iota(jnp.int32, (N, 20), 0)

    hl = _leaky(gcn(_mm(x, wl1_ref[...], _DF), a_l, dis_l, bl1_ref[...]))
    hr = _leaky(gcn(_mm(x, wr1_ref[...], _DF), a_r, dis_r, br1_ref[...]))
    h1 = jnp.where(riota64 < KPOOL, hl, hr)

    hl2 = _leaky(gcn(_mm(h1, wl2_ref[...], _DF), a_l, dis_l, bl2_ref[...]))
    hr2 = _leaky(gcn(_mm(h1, wr2_ref[...], _DF), a_r, dis_r, br2_ref[...]))
    h2a = jnp.where(riota20 < KPOOL, hl2, hr2)

    h2 = _leaky(gcn(_mm(h2a, wg1_ref[...], _DF), a_f, dis_f, bg1_ref[...]))

    agg = _mm(bc_mat, h2, _HI)
    score = jnp.tanh(_mm(agg, wrel_ref[...], _DF) + brel_ref[...]
                     + _mm(h2, wroot_ref[...], _DF))

    score_row = _dot(score, eye, ((0,), (0,)), _HI)
    ri = lax.broadcasted_iota(jnp.int32, (N, N), 0)
    rj = lax.broadcasted_iota(jnp.int32, (N, N), 1)
    beats = ((score_row > score) |
             ((score_row == score) & (rj < ri))).astype(f32)
    rank = jnp.sum(beats, axis=1, keepdims=True)
    rank_row = _dot(rank, eye, ((0,), (0,)), _HI)

    piota = lax.broadcasted_iota(jnp.int32, (KPOOL, N), 0).astype(f32)
    perm_mat = (rank_row == piota).astype(f32)

    vals = _mm(perm_mat, score, _HI)
    x_pool = _mm(perm_mat, h2, _HI) * vals

    craw = _dot(_mm(perm_mat, bc_mat, _HI), perm_mat, ((1,), (1,)), _HI)
    deg_c = jnp.sum(craw, axis=1, keepdims=True)
    dis_c = jnp.where(deg_c > 0, lax.rsqrt(jnp.where(deg_c > 0, deg_c, 1.0)), 0.0)

    def prop_top(z):
        return -(dis_c * _mm(craw, dis_c * z, _HI))

    h2_top = h2[0:KPOOL, :]
    h2_bot = h2[KPOOL:N, :]
    t1_top = prop_top(h2_top)
    t2_top = 2.0 * prop_top(t1_top) - h2_top
    zeros_bot = jnp.zeros((N - KPOOL, 20), f32)
    tx1 = jnp.concatenate([t1_top, zeros_bot], axis=0)
    tx2 = jnp.concatenate([t2_top, -h2_bot], axis=0)

    cheb = (_mm(h2, wc0_ref[...], _DF) + _mm(tx1, wc1_ref[...], _DF)
            + _mm(tx2, wc2_ref[...], _DF) + bc_ref[...])
    ass = _softmax(cheb)
    s = _softmax(ass)

    h_coarse = _dot(s, h2, ((0,), (0,)), _DF)

    kept = (rank < KPOOL).astype(f32)
    tri = (rj < ri).astype(f32)
    cum_excl = _mm(tri, kept, _HI)
    cum_row = _dot(cum_excl, eye, ((0,), (0,)), _HI)
    kept_row = rank_row < KPOOL
    qiota = lax.broadcasted_iota(jnp.int32, (KPOOL, N), 0).astype(f32)
    q_mat = ((cum_row == qiota) & kept_row).astype(f32)

    inter = _mm(q_mat, ass, _HI)
    h1_out = _mm(inter, h_coarse, _DF)
    out_ref[...] = x_pool + h1_out


def kernel(x, edge_index, edge_attr, adj, W_l1, b_l1, W_r1, b_r1, W_l2, b_l2,
           W_r2, b_r2, W_g1, b_g1, W_rel, b_rel, W_root, W_c0, W_c1, W_c2, b_c):
    del adj
    f32 = jnp.float32
    pad = EPAD - E
    acc2 = _densify_sc()(
        jnp.pad(edge_index[0], (0, pad)), jnp.pad(edge_index[1], (0, pad)),
        jnp.pad(edge_attr.astype(f32), (0, pad)),
        jnp.ones((EPAD,), f32),
        jnp.zeros((ACC_FLAT,), f32))
    operands = (
        acc2, x,
        W_l1, b_l1.reshape(1, -1), W_r1, b_r1.reshape(1, -1),
        W_l2, b_l2.reshape(1, -1), W_r2, b_r2.reshape(1, -1),
        W_g1, b_g1.reshape(1, -1), W_rel, b_rel.reshape(1, 1), W_root,
        W_c0, W_c1, W_c2, b_c.reshape(1, -1),
    )
    h2_out = pl.pallas_call(
        _fwd_kernel,
        out_shape=jax.ShapeDtypeStruct((KPOOL, 20), jnp.float32),
    )(*operands)
    return h2_out.reshape(1, -1)
